# baseline (speedup 1.0000x reference)
import functools

import jax
import jax.numpy as jnp
from jax import lax
from jax.experimental import pallas as pl
from jax.experimental.pallas import tpu as pltpu
from jax.experimental.pallas import tpu_sc as plsc

N = 10000
F = 128
E = 320000
NC = 2
NS = 16
FH = F // NC
CH = 40
PER_TILE = E // NS
STEPS = PER_TILE // CH
NBUF = 4
ROWS_PER_SUB = N // NS


def _agg_body(hs_hbm, src_hbm, dst_hbm, out_hbm,
              src_v, dst_v, rows, acc, tab, gsems, ssems):
  c = lax.axis_index("c")
  s = lax.axis_index("s")
  col0 = c * FH
  hview = hs_hbm.at[c]

  row0 = s * ROWS_PER_SUB
  for t in range(16):
    r = row0 + t * CH
    n = CH if t < 15 else ROWS_PER_SUB - 15 * CH
    pltpu.sync_copy(hview.at[pl.ds(r, n)], rows.at[0, pl.ds(0, n)])
    pltpu.sync_copy(rows.at[0, pl.ds(0, n)], tab.at[pl.ds(r, n)])
    pltpu.sync_copy(rows.at[0, pl.ds(0, n)], acc.at[pl.ds(r, n)])
  plsc.subcore_barrier()

  pltpu.sync_copy(src_hbm.at[pl.ds(s * STEPS, STEPS)], src_v)
  pltpu.sync_copy(dst_hbm.at[pl.ds(s * STEPS, STEPS)], dst_v)

  def gather_sp(m, k):
    pltpu.async_copy(tab.at[src_v.at[m]], rows.at[k], gsems.at[k])

  def gather_hbm(m, k):
    pltpu.async_copy(hview.at[src_v.at[m]], rows.at[k], gsems.at[k])

  def gather(m, k, i):
    if k == 3:
      gather_sp(m, k)
    elif k == 1:
      p = (m // NBUF) % 4 == 3
      @pl.when(p)
      def _():
        gather_sp(m, k)
      @pl.when(jnp.logical_not(p))
      def _():
        gather_hbm(m, k)
    else:
      gather_hbm(m, k)

  def scatter_start(m, k):
    pltpu.async_copy(rows.at[k], acc.at[dst_v.at[m]], ssems.at[k], add=True)

  def gwait(m, k):
    pltpu.make_async_copy(hview.at[src_v.at[m]], rows.at[k], gsems.at[k]).wait()

  def swait(m, k):
    pltpu.make_async_copy(rows.at[k], acc.at[dst_v.at[m]], ssems.at[k]).wait()

  gather_hbm(0, 0)
  gather_hbm(1, 1)

  def body(i, carry):
    m0 = i * NBUF
    for k in range(NBUF):
      m = m0 + k
      @pl.when(m - 2 >= 0)
      def _():
        swait(m - 2, (k + 2) % NBUF)
      @pl.when(m + 2 < STEPS)
      def _():
        gather(m + 2, (k + 2) % NBUF, i)
      gwait(m, k)
      scatter_start(m, k)
    return carry

  lax.fori_loop(0, STEPS // NBUF, body, 0)
  swait(STEPS - 2, (STEPS - 2) % NBUF)
  swait(STEPS - 1, (STEPS - 1) % NBUF)

  plsc.subcore_barrier()
  for t in range(16):
    r = row0 + t * CH
    n = CH if t < 15 else ROWS_PER_SUB - 15 * CH
    pltpu.sync_copy(acc.at[pl.ds(r, n)], rows.at[0, pl.ds(0, n)])
    pltpu.sync_copy(rows.at[0, pl.ds(0, n)],
                    out_hbm.at[pl.ds(r, n), pl.ds(col0, FH)])


_agg = pl.kernel(
    _agg_body,
    out_type=jax.ShapeDtypeStruct((N, F), jnp.float32),
    mesh=plsc.VectorSubcoreMesh(core_axis_name="c", subcore_axis_name="s"),
    scratch_types=[
        pltpu.VMEM((STEPS, CH), jnp.int32),
        pltpu.VMEM((STEPS, CH), jnp.int32),
        pltpu.VMEM((NBUF, CH, FH), jnp.float32),
        pltpu.VMEM_SHARED((N, FH), jnp.float32),
        pltpu.VMEM_SHARED((N, FH), jnp.float32),
        pltpu.SemaphoreType.DMA((NBUF,)),
        pltpu.SemaphoreType.DMA((NBUF,)),
    ],
    compiler_params=pltpu.CompilerParams(use_tc_tiling_on_sc=False),
)


def _mlp_bn_body(z, Wa, ba, Wb, bb, g, be, out):
  u = jnp.maximum(jnp.dot(z[...], Wa[...], preferred_element_type=jnp.float32)
                  + ba[...], 0.0)
  v = jnp.dot(u, Wb[...], preferred_element_type=jnp.float32) + bb[...]
  v = jnp.maximum(v, 0.0)
  m = jnp.mean(v, axis=0, keepdims=True)
  var = jnp.mean((v - m) * (v - m), axis=0, keepdims=True)
  bn = (v - m) * lax.rsqrt(var + 1e-5) * g[...] + be[...]
  if out.shape[0] == NC:
    out[0] = bn[:, :FH]
    out[1] = bn[:, FH:]
  else:
    out[...] = bn


def _mlp_bn(z, Wa, ba, Wb, bb, g, be, split):
  shape = (NC, N, FH) if split else (N, 2)
  return pl.pallas_call(
      _mlp_bn_body,
      out_shape=jax.ShapeDtypeStruct(shape, jnp.float32),
  )(z, Wa, ba, Wb, bb, g, be)


def kernel(x, edge_index, W1a, b1a, W1b, b1b, g1, be1,
           W2a, b2a, W2b, b2b, g2, be2):
  src2 = edge_index[0].reshape(NS * STEPS, CH)
  dst2 = edge_index[1].reshape(NS * STEPS, CH)
  xs = jnp.stack((x[:, :FH], x[:, FH:]))
  z1 = _agg(xs, src2, dst2)
  h1s = _mlp_bn(z1, W1a, b1a, W1b, b1b, g1, be1, split=True)
  z2 = _agg(h1s, src2, dst2)
  h2 = _mlp_bn(z2, W2a, b2a, W2b, b2b, g2, be2, split=False)
  return h2


# speedup vs baseline: 1.0444x; 1.0444x over previous
import functools

import jax
import jax.numpy as jnp
from jax import lax
from jax.experimental import pallas as pl
from jax.experimental.pallas import tpu as pltpu
from jax.experimental.pallas import tpu_sc as plsc

N = 10000
F = 128
E = 320000
NC = 2
NS = 16
FH = F // NC
CH = 40
PER_TILE = E // NS
STEPS = PER_TILE // CH
NBUF = 4
ROWS_PER_SUB = N // NS


def _agg_body(hs_hbm, src_hbm, dst_hbm, out_hbm,
              src_v, dst_v, rows, acc, tab, gsems, ssems):
  c = lax.axis_index("c")
  s = lax.axis_index("s")
  col0 = c * FH
  hview = hs_hbm.at[c]

  row0 = s * ROWS_PER_SUB
  for t in range(16):
    r = row0 + t * CH
    n = CH if t < 15 else ROWS_PER_SUB - 15 * CH
    pltpu.sync_copy(hview.at[pl.ds(r, n)], rows.at[0, pl.ds(0, n)])
    pltpu.sync_copy(rows.at[0, pl.ds(0, n)], tab.at[pl.ds(r, n)])
    pltpu.sync_copy(rows.at[0, pl.ds(0, n)], acc.at[pl.ds(r, n)])
  plsc.subcore_barrier()

  pltpu.sync_copy(src_hbm.at[pl.ds(s * STEPS, STEPS)], src_v)
  pltpu.sync_copy(dst_hbm.at[pl.ds(s * STEPS, STEPS)], dst_v)

  def gather_sp(m, k):
    pltpu.async_copy(tab.at[src_v.at[m]], rows.at[k], gsems.at[k])

  def gather_hbm(m, k):
    pltpu.async_copy(hview.at[src_v.at[m]], rows.at[k], gsems.at[k])

  def gather(m, k, i):
    if k % 2 == 0:
      gather_sp(m, k)
    else:
      gather_hbm(m, k)

  def scatter_start(m, k):
    pltpu.async_copy(rows.at[k], acc.at[dst_v.at[m]], ssems.at[k], add=True)

  def gwait(m, k):
    pltpu.make_async_copy(hview.at[src_v.at[m]], rows.at[k], gsems.at[k]).wait()

  def swait(m, k):
    pltpu.make_async_copy(rows.at[k], acc.at[dst_v.at[m]], ssems.at[k]).wait()

  gather_sp(0, 0)
  gather_hbm(1, 1)

  def body(i, carry):
    m0 = i * NBUF
    for k in range(NBUF):
      m = m0 + k
      @pl.when(m - 2 >= 0)
      def _():
        swait(m - 2, (k + 2) % NBUF)
      @pl.when(m + 2 < STEPS)
      def _():
        gather(m + 2, (k + 2) % NBUF, i)
      gwait(m, k)
      scatter_start(m, k)
    return carry

  lax.fori_loop(0, STEPS // NBUF, body, 0)
  swait(STEPS - 2, (STEPS - 2) % NBUF)
  swait(STEPS - 1, (STEPS - 1) % NBUF)

  plsc.subcore_barrier()
  for t in range(16):
    r = row0 + t * CH
    n = CH if t < 15 else ROWS_PER_SUB - 15 * CH
    pltpu.sync_copy(acc.at[pl.ds(r, n)], rows.at[0, pl.ds(0, n)])
    pltpu.sync_copy(rows.at[0, pl.ds(0, n)],
                    out_hbm.at[pl.ds(r, n), pl.ds(col0, FH)])


_agg = pl.kernel(
    _agg_body,
    out_type=jax.ShapeDtypeStruct((N, F), jnp.float32),
    mesh=plsc.VectorSubcoreMesh(core_axis_name="c", subcore_axis_name="s"),
    scratch_types=[
        pltpu.VMEM((STEPS, CH), jnp.int32),
        pltpu.VMEM((STEPS, CH), jnp.int32),
        pltpu.VMEM((NBUF, CH, FH), jnp.float32),
        pltpu.VMEM_SHARED((N, FH), jnp.float32),
        pltpu.VMEM_SHARED((N, FH), jnp.float32),
        pltpu.SemaphoreType.DMA((NBUF,)),
        pltpu.SemaphoreType.DMA((NBUF,)),
    ],
    compiler_params=pltpu.CompilerParams(use_tc_tiling_on_sc=False),
)


def _mlp_bn_body(z, Wa, ba, Wb, bb, g, be, out):
  u = jnp.maximum(jnp.dot(z[...], Wa[...], preferred_element_type=jnp.float32)
                  + ba[...], 0.0)
  v = jnp.dot(u, Wb[...], preferred_element_type=jnp.float32) + bb[...]
  v = jnp.maximum(v, 0.0)
  m = jnp.mean(v, axis=0, keepdims=True)
  var = jnp.mean((v - m) * (v - m), axis=0, keepdims=True)
  bn = (v - m) * lax.rsqrt(var + 1e-5) * g[...] + be[...]
  if out.shape[0] == NC:
    out[0] = bn[:, :FH]
    out[1] = bn[:, FH:]
  else:
    out[...] = bn


def _mlp_bn(z, Wa, ba, Wb, bb, g, be, split):
  shape = (NC, N, FH) if split else (N, 2)
  return pl.pallas_call(
      _mlp_bn_body,
      out_shape=jax.ShapeDtypeStruct(shape, jnp.float32),
  )(z, Wa, ba, Wb, bb, g, be)


def kernel(x, edge_index, W1a, b1a, W1b, b1b, g1, be1,
           W2a, b2a, W2b, b2b, g2, be2):
  src2 = edge_index[0].reshape(NS * STEPS, CH)
  dst2 = edge_index[1].reshape(NS * STEPS, CH)
  xs = jnp.stack((x[:, :FH], x[:, FH:]))
  z1 = _agg(xs, src2, dst2)
  h1s = _mlp_bn(z1, W1a, b1a, W1b, b1b, g1, be1, split=True)
  z2 = _agg(h1s, src2, dst2)
  h2 = _mlp_bn(z2, W2a, b2a, W2b, b2b, g2, be2, split=False)
  return h2


# speedup vs baseline: 1.2519x; 1.1987x over previous
import functools

import jax
import jax.numpy as jnp
from jax import lax
from jax.experimental import pallas as pl
from jax.experimental.pallas import tpu as pltpu
from jax.experimental.pallas import tpu_sc as plsc

N = 10000
F = 128
E = 320000
NC = 2
NS = 16
FH = F // NC
CH = 100
PER_TILE = E // NS
STEPS = PER_TILE // CH
HSTEPS = STEPS // 2
NBUF = 4
ROWS_PER_SUB = N // NS


def _agg_body(h_hbm, src_hbm, dst_hbm, out_hbm,
              src_v, dst_v, rows, acc, tab, gsems, ssems):
  c = lax.axis_index("c")
  s = lax.axis_index("s")
  col0 = c * FH

  row0 = s * ROWS_PER_SUB
  for t in range(7):
    r = row0 + t * CH
    n = CH if t < 6 else ROWS_PER_SUB - 6 * CH
    pltpu.sync_copy(h_hbm.at[pl.ds(r, n), pl.ds(col0, FH)],
                    rows.at[0, pl.ds(0, n)])
    pltpu.sync_copy(rows.at[0, pl.ds(0, n)], tab.at[pl.ds(r, n)])
    pltpu.sync_copy(rows.at[0, pl.ds(0, n)], acc.at[pl.ds(r, n)])
  plsc.subcore_barrier()


  def gather(m, k):
    pltpu.async_copy(tab.at[src_v.at[m]], rows.at[k], gsems.at[k])

  def scatter_start(m, k):
    pltpu.async_copy(rows.at[k], acc.at[dst_v.at[m]], ssems.at[k], add=True)

  def gwait(m, k):
    pltpu.make_async_copy(tab.at[src_v.at[m]], rows.at[k], gsems.at[k]).wait()

  def swait(m, k):
    pltpu.make_async_copy(rows.at[k], acc.at[dst_v.at[m]], ssems.at[k]).wait()

  def body(i, carry):
    m0 = i * NBUF
    for k in range(NBUF):
      m = m0 + k
      @pl.when(m - 2 >= 0)
      def _():
        swait(m - 2, (k + 2) % NBUF)
      @pl.when(m + 2 < HSTEPS)
      def _():
        gather(m + 2, (k + 2) % NBUF)
      gwait(m, k)
      scatter_start(m, k)
    return carry

  for half in range(2):
    base = s * STEPS + half * HSTEPS
    pltpu.sync_copy(src_hbm.at[pl.ds(base, HSTEPS)], src_v)
    pltpu.sync_copy(dst_hbm.at[pl.ds(base, HSTEPS)], dst_v)
    gather(0, 0)
    gather(1, 1)
    lax.fori_loop(0, HSTEPS // NBUF, body, 0)
    swait(HSTEPS - 2, (HSTEPS - 2) % NBUF)
    swait(HSTEPS - 1, (HSTEPS - 1) % NBUF)

  plsc.subcore_barrier()
  for t in range(7):
    r = row0 + t * CH
    n = CH if t < 6 else ROWS_PER_SUB - 6 * CH
    pltpu.sync_copy(acc.at[pl.ds(r, n)], rows.at[0, pl.ds(0, n)])
    pltpu.sync_copy(rows.at[0, pl.ds(0, n)],
                    out_hbm.at[pl.ds(r, n), pl.ds(col0, FH)])


_agg = pl.kernel(
    _agg_body,
    out_type=jax.ShapeDtypeStruct((N, F), jnp.float32),
    mesh=plsc.VectorSubcoreMesh(core_axis_name="c", subcore_axis_name="s"),
    scratch_types=[
        pltpu.VMEM((HSTEPS, CH), jnp.int32),
        pltpu.VMEM((HSTEPS, CH), jnp.int32),
        pltpu.VMEM((NBUF, CH, FH), jnp.float32),
        pltpu.VMEM_SHARED((N, FH), jnp.float32),
        pltpu.VMEM_SHARED((N, FH), jnp.float32),
        pltpu.SemaphoreType.DMA((NBUF,)),
        pltpu.SemaphoreType.DMA((NBUF,)),
    ],
    compiler_params=pltpu.CompilerParams(use_tc_tiling_on_sc=False),
)


def _mlp_bn_body(z, Wa, ba, Wb, bb, g, be, out):
  u = jnp.maximum(jnp.dot(z[...], Wa[...], preferred_element_type=jnp.float32)
                  + ba[...], 0.0)
  v = jnp.dot(u, Wb[...], preferred_element_type=jnp.float32) + bb[...]
  v = jnp.maximum(v, 0.0)
  m = jnp.mean(v, axis=0, keepdims=True)
  var = jnp.mean((v - m) * (v - m), axis=0, keepdims=True)
  out[...] = (v - m) * lax.rsqrt(var + 1e-5) * g[...] + be[...]


def _mlp_bn(z, Wa, ba, Wb, bb, g, be, dout):
  return pl.pallas_call(
      _mlp_bn_body,
      out_shape=jax.ShapeDtypeStruct((N, dout), jnp.float32),
  )(z, Wa, ba, Wb, bb, g, be)


def kernel(x, edge_index, W1a, b1a, W1b, b1b, g1, be1,
           W2a, b2a, W2b, b2b, g2, be2):
  src2 = edge_index[0].reshape(NS * STEPS, CH)
  dst2 = edge_index[1].reshape(NS * STEPS, CH)
  z1 = _agg(x, src2, dst2)
  h1 = _mlp_bn(z1, W1a, b1a, W1b, b1b, g1, be1, F)
  z2 = _agg(h1, src2, dst2)
  h2 = _mlp_bn(z2, W2a, b2a, W2b, b2b, g2, be2, 2)
  return h2


# speedup vs baseline: 1.2531x; 1.0009x over previous
import jax
import jax.numpy as jnp
from jax import lax
from jax.experimental import pallas as pl
from jax.experimental.pallas import tpu as pltpu
from jax.experimental.pallas import tpu_sc as plsc

N = 10000
F = 128
E = 320000
NC = 2
NS = 16
FH = F // NC
CH = 100
PER_TILE = E // NS
STEPS = PER_TILE // CH
HSTEPS = STEPS // 2
NBUF = 4
ROWS_PER_SUB = N // NS


def _agg_body(h_hbm, src_hbm, dst_hbm, out_hbm,
              src_v, dst_v, rows, acc, tab, gsems, ssems):
  c = lax.axis_index("c")
  s = lax.axis_index("s")
  col0 = c * FH

  row0 = s * ROWS_PER_SUB
  for t in range(7):
    r = row0 + t * CH
    n = CH if t < 6 else ROWS_PER_SUB - 6 * CH
    pltpu.sync_copy(h_hbm.at[pl.ds(r, n), pl.ds(col0, FH)],
                    rows.at[0, pl.ds(0, n)])
    pltpu.sync_copy(rows.at[0, pl.ds(0, n)], tab.at[pl.ds(r, n)])
    pltpu.sync_copy(rows.at[0, pl.ds(0, n)], acc.at[pl.ds(r, n)])
  plsc.subcore_barrier()


  def gather(m, k):
    pltpu.async_copy(tab.at[src_v.at[m]], rows.at[k], gsems.at[k])

  def scatter_start(m, k):
    pltpu.async_copy(rows.at[k], acc.at[dst_v.at[m]], ssems.at[k], add=True)

  def gwait(m, k):
    pltpu.make_async_copy(tab.at[src_v.at[m]], rows.at[k], gsems.at[k]).wait()

  def swait(m, k):
    pltpu.make_async_copy(rows.at[k], acc.at[dst_v.at[m]], ssems.at[k]).wait()

  def body(i, carry):
    m0 = i * NBUF
    for k in range(NBUF):
      m = m0 + k
      @pl.when(m - 2 >= 0)
      def _():
        swait(m - 2, (k + 2) % NBUF)
      @pl.when(m + 2 < HSTEPS)
      def _():
        gather(m + 2, (k + 2) % NBUF)
      gwait(m, k)
      scatter_start(m, k)
    return carry

  for half in range(2):
    base = s * STEPS + half * HSTEPS
    pltpu.sync_copy(src_hbm.at[pl.ds(base, HSTEPS)], src_v)
    pltpu.sync_copy(dst_hbm.at[pl.ds(base, HSTEPS)], dst_v)
    gather(0, 0)
    gather(1, 1)
    lax.fori_loop(0, HSTEPS // NBUF, body, 0)
    swait(HSTEPS - 2, (HSTEPS - 2) % NBUF)
    swait(HSTEPS - 1, (HSTEPS - 1) % NBUF)

  plsc.subcore_barrier()
  for t in range(7):
    r = row0 + t * CH
    n = CH if t < 6 else ROWS_PER_SUB - 6 * CH
    pltpu.sync_copy(acc.at[pl.ds(r, n)], rows.at[0, pl.ds(0, n)])
    pltpu.sync_copy(rows.at[0, pl.ds(0, n)],
                    out_hbm.at[pl.ds(r, n), pl.ds(col0, FH)])


_agg = pl.kernel(
    _agg_body,
    out_type=jax.ShapeDtypeStruct((N, F), jnp.float32),
    mesh=plsc.VectorSubcoreMesh(core_axis_name="c", subcore_axis_name="s"),
    scratch_types=[
        pltpu.VMEM((HSTEPS, CH), jnp.int32),
        pltpu.VMEM((HSTEPS, CH), jnp.int32),
        pltpu.VMEM((NBUF, CH, FH), jnp.float32),
        pltpu.VMEM_SHARED((N, FH), jnp.float32),
        pltpu.VMEM_SHARED((N, FH), jnp.float32),
        pltpu.SemaphoreType.DMA((NBUF,)),
        pltpu.SemaphoreType.DMA((NBUF,)),
    ],
    compiler_params=pltpu.CompilerParams(use_tc_tiling_on_sc=False),
)


def _mlp_bn_body(z, Wa, ba, Wb, bb, g, be, out):
  u = jnp.maximum(jnp.dot(z[...], Wa[...], preferred_element_type=jnp.float32)
                  + ba[...], 0.0)
  v = jnp.dot(u, Wb[...], preferred_element_type=jnp.float32) + bb[...]
  v = jnp.maximum(v, 0.0)
  m = jnp.mean(v, axis=0, keepdims=True)
  var = jnp.mean((v - m) * (v - m), axis=0, keepdims=True)
  out[...] = (v - m) * lax.rsqrt(var + 1e-5) * g[...] + be[...]


def _mlp_bn(z, Wa, ba, Wb, bb, g, be, dout):
  return pl.pallas_call(
      _mlp_bn_body,
      out_shape=jax.ShapeDtypeStruct((N, dout), jnp.float32),
  )(z, Wa, ba, Wb, bb, g, be)


def kernel(x, edge_index, W1a, b1a, W1b, b1b, g1, be1,
           W2a, b2a, W2b, b2b, g2, be2):
  src2 = edge_index[0].reshape(NS * STEPS, CH)
  dst2 = edge_index[1].reshape(NS * STEPS, CH)
  z1 = _agg(x, src2, dst2)
  h1 = _mlp_bn(z1, W1a, b1a, W1b, b1b, g1, be1, F)
  z2 = _agg(h1, src2, dst2)
  h2 = _mlp_bn(z2, W2a, b2a, W2b, b2b, g2, be2, 2)
  return h2
